# baseline (speedup 1.0000x reference)
import functools

import jax
import jax.numpy as jnp
from jax import lax
from jax.experimental import pallas as pl
from jax.experimental.pallas import tpu as pltpu
from jax.experimental.pallas import tpu_sc as plsc

N = 10000
NP = 10240
E = 320000
H = 128
NB = 512
GN = NP // NB
EBLK = 1280
GE = E // EBLK

NC = 2
NS = 16
NW = NC * NS
P = 2
EPARTS = (163840, 156160)
CHM = 40
CHG = 80
MSTAGE = 2
RPT = NP // NS


def _matmul_bias_kernel(x_ref, w_ref, b_ref, o_ref):
    o_ref[...] = (
        jnp.dot(x_ref[...], w_ref[...], preferred_element_type=jnp.float32)
        + b_ref[...]
    )


def _matmul_bias(x, W, b, blk):
    M, K = x.shape
    Ho = W.shape[1]
    return pl.pallas_call(
        _matmul_bias_kernel,
        grid=(M // blk,),
        in_specs=[
            pl.BlockSpec((blk, K), lambda i: (i, 0)),
            pl.BlockSpec((K, Ho), lambda i: (0, 0)),
            pl.BlockSpec((1, Ho), lambda i: (0, 0)),
        ],
        out_specs=pl.BlockSpec((blk, Ho), lambda i: (i, 0)),
        out_shape=jax.ShapeDtypeStruct((M, Ho), jnp.float32),
    )(x, W, b.reshape(1, Ho))


def _node_z_kernel(h_ref, p0_ref, p1_ref, p2_ref, p3_ref,
                   w1_ref, b1_ref, w2_ref, b2_ref,
                   epsb_ref, z_ref, s_ref, s2_ref):
    i = pl.program_id(0)
    z = (epsb_ref[...] * h_ref[...] + (p0_ref[...] + p1_ref[...])
         + (p2_ref[...] + p3_ref[...]))
    z = jnp.maximum(
        jnp.dot(z, w1_ref[...], preferred_element_type=jnp.float32)
        + b1_ref[...], 0.0)
    z = jnp.dot(z, w2_ref[...], preferred_element_type=jnp.float32) + b2_ref[...]
    z_ref[...] = z
    rows = i * NB + lax.broadcasted_iota(jnp.int32, (NB, 1), 0)
    zm = jnp.where(rows < N, z, 0.0)
    s_ref[...] = jnp.sum(zm, axis=0, keepdims=True)[None]
    s2_ref[...] = jnp.sum(zm * zm, axis=0, keepdims=True)[None]


def _node_z(h, p0, p1, p2, p3, W1l, b1l, W2l, b2l, epsb):
    return pl.pallas_call(
        _node_z_kernel,
        grid=(GN,),
        in_specs=[
            pl.BlockSpec((NB, H), lambda i: (i, 0)),
            pl.BlockSpec((NB, H), lambda i: (i, 0)),
            pl.BlockSpec((NB, H), lambda i: (i, 0)),
            pl.BlockSpec((NB, H), lambda i: (i, 0)),
            pl.BlockSpec((NB, H), lambda i: (i, 0)),
            pl.BlockSpec((H, H), lambda i: (0, 0)),
            pl.BlockSpec((1, H), lambda i: (0, 0)),
            pl.BlockSpec((H, H), lambda i: (0, 0)),
            pl.BlockSpec((1, H), lambda i: (0, 0)),
            pl.BlockSpec((1, H), lambda i: (0, 0)),
        ],
        out_specs=[
            pl.BlockSpec((NB, H), lambda i: (i, 0)),
            pl.BlockSpec((1, 1, H), lambda i: (i, 0, 0)),
            pl.BlockSpec((1, 1, H), lambda i: (i, 0, 0)),
        ],
        out_shape=[
            jax.ShapeDtypeStruct((NP, H), jnp.float32),
            jax.ShapeDtypeStruct((GN, 1, H), jnp.float32),
            jax.ShapeDtypeStruct((GN, 1, H), jnp.float32),
        ],
    )(h, p0, p1, p2, p3, W1l, b1l.reshape(1, H), W2l, b2l.reshape(1, H), epsb)


def _bn_update(z_ref, h_ref, s_ref, s2_ref, g_ref, be_ref):
    S = jnp.sum(s_ref[...], axis=0)
    S2 = jnp.sum(s2_ref[...], axis=0)
    mu = S * (1.0 / N)
    var = S2 * (1.0 / N) - mu * mu
    inv = lax.rsqrt(var + 1e-5)
    zbn = g_ref[...] * (z_ref[...] - mu) * inv + be_ref[...]
    return (h_ref[...] + jnp.maximum(zbn, 0.0)) * 0.5


def _node_bn_ab_kernel(z_ref, h_ref, s_ref, s2_ref, g_ref, be_ref,
                       wa_ref, wb_ref, h_out, a_out, b_out):
    hn = _bn_update(z_ref, h_ref, s_ref, s2_ref, g_ref, be_ref)
    h_out[...] = hn
    a_out[...] = jnp.dot(hn, wa_ref[...], preferred_element_type=jnp.float32)
    b_out[...] = jnp.dot(hn, wb_ref[...], preferred_element_type=jnp.float32)


def _node_bn_ab(z, h, s, s2, gl, bl, Wa, Wb):
    return pl.pallas_call(
        _node_bn_ab_kernel,
        grid=(GN,),
        in_specs=[
            pl.BlockSpec((NB, H), lambda i: (i, 0)),
            pl.BlockSpec((NB, H), lambda i: (i, 0)),
            pl.BlockSpec((GN, 1, H), lambda i: (0, 0, 0)),
            pl.BlockSpec((GN, 1, H), lambda i: (0, 0, 0)),
            pl.BlockSpec((1, H), lambda i: (0, 0)),
            pl.BlockSpec((1, H), lambda i: (0, 0)),
            pl.BlockSpec((H, H), lambda i: (0, 0)),
            pl.BlockSpec((H, H), lambda i: (0, 0)),
        ],
        out_specs=[
            pl.BlockSpec((NB, H), lambda i: (i, 0)),
            pl.BlockSpec((NB, H), lambda i: (i, 0)),
            pl.BlockSpec((NB, H), lambda i: (i, 0)),
        ],
        out_shape=[
            jax.ShapeDtypeStruct((NP, H), jnp.float32),
            jax.ShapeDtypeStruct((NP, H), jnp.float32),
            jax.ShapeDtypeStruct((NP, H), jnp.float32),
        ],
    )(z, h, s, s2, gl.reshape(1, H), bl.reshape(1, H), Wa, Wb)


def _node_bn_head_kernel(z_ref, h_ref, s_ref, s2_ref, g_ref, be_ref,
                         wh_ref, bh_ref, p_out):
    hn = _bn_update(z_ref, h_ref, s_ref, s2_ref, g_ref, be_ref)
    p_out[...] = (
        jnp.dot(hn, wh_ref[...], preferred_element_type=jnp.float32)
        + bh_ref[...]
    )


def _node_bn_head(z, h, s, s2, gl, bl, Whp, bhp):
    return pl.pallas_call(
        _node_bn_head_kernel,
        grid=(GN,),
        in_specs=[
            pl.BlockSpec((NB, H), lambda i: (i, 0)),
            pl.BlockSpec((NB, H), lambda i: (i, 0)),
            pl.BlockSpec((GN, 1, H), lambda i: (0, 0, 0)),
            pl.BlockSpec((GN, 1, H), lambda i: (0, 0, 0)),
            pl.BlockSpec((1, H), lambda i: (0, 0)),
            pl.BlockSpec((1, H), lambda i: (0, 0)),
            pl.BlockSpec((H, H), lambda i: (0, 0)),
            pl.BlockSpec((1, H), lambda i: (0, 0)),
        ],
        out_specs=pl.BlockSpec((NB, H), lambda i: (i, 0)),
        out_shape=jax.ShapeDtypeStruct((NP, H), jnp.float32),
    )(z, h, s, s2, gl.reshape(1, H), bl.reshape(1, H), Whp, bhp)


def _edge_mlp_kernel(e_ref, g_ref, we_ref, b1_ref, w2_ref, b2_ref, o_ref):
    e = e_ref[...]
    hid = jnp.maximum(
        jnp.dot(e, we_ref[...], preferred_element_type=jnp.float32)
        + g_ref[...] + b1_ref[...], 0.0)
    upd = jnp.dot(hid, w2_ref[...], preferred_element_type=jnp.float32) + b2_ref[...]
    o_ref[...] = e + 0.5 * upd


def _edge_mlp(e, G, We1e, be1l, We2l, be2l):
    ep = e.shape[0]
    return pl.pallas_call(
        _edge_mlp_kernel,
        grid=(ep // EBLK,),
        in_specs=[
            pl.BlockSpec((EBLK, H), lambda i: (i, 0)),
            pl.BlockSpec((EBLK, H), lambda i: (i, 0)),
            pl.BlockSpec((H, H), lambda i: (0, 0)),
            pl.BlockSpec((1, H), lambda i: (0, 0)),
            pl.BlockSpec((H, H), lambda i: (0, 0)),
            pl.BlockSpec((1, H), lambda i: (0, 0)),
        ],
        out_specs=pl.BlockSpec((EBLK, H), lambda i: (i, 0)),
        out_shape=jax.ShapeDtypeStruct((ep, H), jnp.float32),
    )(e, G, We1e, be1l.reshape(1, H), We2l, be2l.reshape(1, H))


def _sc_mesh():
    return plsc.VectorSubcoreMesh(core_axis_name="c", subcore_axis_name="s")


def _relu_add_rows(rows_v, ev_v, n_rows):
    def rbody(g, c2):
        r = 2 * g
        for rr in range(2):
            for cc in range(H // 16):
                sl = pl.ds(cc * 16, 16)
                rows_v[r + rr, sl] = jnp.maximum(
                    rows_v[r + rr, sl] + ev_v[r + rr, sl], 0.0)
        return c2

    lax.fori_loop(0, n_rows // 2, rbody, 0)


def _add_rows(ra_v, rb_v, n_rows):
    def rbody(g, c2):
        r = 2 * g
        for rr in range(2):
            for cc in range(H // 16):
                sl = pl.ds(cc * 16, 16)
                ra_v[r + rr, sl] = ra_v[r + rr, sl] + rb_v[r + rr, sl]
        return c2

    lax.fori_loop(0, n_rows // 2, rbody, 0)


def _msg_scatter(h, e, src4, dst4, zeros, epw, stage_counts, stage_cap):
    n_stages = len(stage_counts)

    @functools.partial(
        pl.kernel,
        mesh=_sc_mesh(),
        out_type=jax.ShapeDtypeStruct((NC, NP, H), jnp.float32),
        scratch_types=[
            pltpu.VMEM((stage_cap, CHM), jnp.int32),
            pltpu.VMEM((stage_cap, CHM), jnp.int32),
            pltpu.VMEM((CHM, H), jnp.float32),
            pltpu.VMEM((CHM, H), jnp.float32),
            pltpu.VMEM((CHM, H), jnp.float32),
            pltpu.VMEM((CHM, H), jnp.float32),
            pltpu.VMEM_SHARED((NP, H), jnp.float32),
            pltpu.SemaphoreType.DMA,
            pltpu.SemaphoreType.DMA,
            pltpu.SemaphoreType.DMA,
            pltpu.SemaphoreType.DMA,
            pltpu.SemaphoreType.DMA,
            pltpu.SemaphoreType.DMA,
            pltpu.SemaphoreType.DMA,
        ],
    )
    def k(h_hbm, e_hbm, src_hbm, dst_hbm, z_hbm, out_hbm,
          srcb, dstb, rows0, rows1, rows2, evb, acc,
          sg0, sg1, sg2, se, ss0, ss1, ss2):
        c = lax.axis_index("c")
        s = lax.axis_index("s")
        wid = s * NC + c
        base = wid * epw
        rows = (rows0, rows1, rows2)
        sg = (sg0, sg1, sg2)
        ss = (ss0, ss1, ss2)

        pltpu.sync_copy(z_hbm.at[pl.ds(s * RPT, RPT)],
                        acc.at[pl.ds(s * RPT, RPT)])
        plsc.subcore_barrier()

        for part in range(n_stages):
            M = stage_counts[part]
            hoff = sum(stage_counts[:part])
            pltpu.sync_copy(src_hbm.at[wid, part], srcb)
            pltpu.sync_copy(dst_hbm.at[wid, part], dstb)

            def start_g(j, b):
                pltpu.async_copy(h_hbm.at[srcb.at[j]], rows[b], sg[b])

            def finish_g(j, b):
                pltpu.make_async_copy(h_hbm.at[srcb.at[j]], rows[b],
                                      sg[b]).wait()

            def start_e(j):
                pltpu.async_copy(
                    e_hbm.at[pl.ds(base + (hoff + j) * CHM, CHM)], evb, se)

            def finish_e(j):
                pltpu.make_async_copy(
                    e_hbm.at[pl.ds(base + (hoff + j) * CHM, CHM)],
                    evb, se).wait()

            def scat_start(j, b):
                pltpu.async_copy(rows[b], acc.at[dstb.at[j]], ss[b],
                                 add=True)

            def scat_wait(j, b):
                pltpu.make_async_copy(rows[b], acc.at[dstb.at[j]],
                                      ss[b]).wait()

            def step(j, b):
                finish_g(j, b)
                finish_e(j)
                _relu_add_rows(rows[b], evb, CHM)

                @pl.when(j + 1 < M)
                def _():
                    start_e(j + 1)

                scat_start(j, b)
                bn = (b + 2) % 3

                @pl.when((j + 2 < M) & (j >= 1))
                def _():
                    scat_wait(j - 1, bn)

                @pl.when(j + 2 < M)
                def _():
                    start_g(j + 2, bn)

            start_g(0, 0)
            start_g(1, 1)
            start_e(0)

            def body(g, carry):
                step(3 * g, 0)
                step(3 * g + 1, 1)
                step(3 * g + 2, 2)
                return carry

            lax.fori_loop(0, M // 3, body, 0)
            for j in range(3 * (M // 3), M):
                step(j, j % 3)
            for j in range(max(0, M - 3), M):
                scat_wait(j, j % 3)

        plsc.subcore_barrier()
        pltpu.sync_copy(acc.at[pl.ds(s * RPT, RPT)],
                        out_hbm.at[c, pl.ds(s * RPT, RPT)])

    return k(h, e, src4, dst4, zeros)


def _gather_ab(A, B, src3, dst3, ep, epw, nch):
    @functools.partial(
        pl.kernel,
        mesh=_sc_mesh(),
        out_type=jax.ShapeDtypeStruct((ep, H), jnp.float32),
        scratch_types=[
            pltpu.VMEM((nch, CHG), jnp.int32),
            pltpu.VMEM((nch, CHG), jnp.int32),
            pltpu.VMEM((CHG, H), jnp.float32),
            pltpu.VMEM((CHG, H), jnp.float32),
            pltpu.VMEM((CHG, H), jnp.float32),
            pltpu.VMEM((CHG, H), jnp.float32),
            pltpu.SemaphoreType.DMA,
            pltpu.SemaphoreType.DMA,
            pltpu.SemaphoreType.DMA,
            pltpu.SemaphoreType.DMA,
        ],
    )
    def k(a_hbm, b_hbm, src_hbm, dst_hbm, out_hbm,
          srcb, dstb, ra0, ra1, rb0, rb1, sa0, sa1, sb0, sb1):
        c = lax.axis_index("c")
        s = lax.axis_index("s")
        wid = s * NC + c
        base = wid * epw
        ra = (ra0, ra1)
        rb = (rb0, rb1)
        sa = (sa0, sa1)
        sb = (sb0, sb1)

        pltpu.sync_copy(src_hbm.at[wid], srcb)
        pltpu.sync_copy(dst_hbm.at[wid], dstb)

        def start(j, b):
            pltpu.async_copy(a_hbm.at[srcb.at[j]], ra[b], sa[b])
            pltpu.async_copy(b_hbm.at[dstb.at[j]], rb[b], sb[b])

        def finish(j, b):
            pltpu.make_async_copy(a_hbm.at[srcb.at[j]], ra[b], sa[b]).wait()
            pltpu.make_async_copy(b_hbm.at[dstb.at[j]], rb[b], sb[b]).wait()

        def step(j, b):
            finish(j, b)
            _add_rows(ra[b], rb[b], CHG)
            pltpu.sync_copy(ra[b], out_hbm.at[pl.ds(base + j * CHG, CHG)])
            jn = j + 2

            @pl.when(jn < nch)
            def _():
                start(jn, b)

        start(0, 0)
        start(1, 1)

        def body(g, carry):
            step(2 * g, 0)
            step(2 * g + 1, 1)
            return carry

        lax.fori_loop(0, nch // 2, body, 0)
        if nch % 2:
            step(nch - 1, 0)

    return k(A, B, src3, dst3)


def kernel(x, edge_index, edge_attr, Wn, bn, We, be, eps, W1, b1, W2, b2,
           gamma, beta, We1, be1, We2, be2, Wh, bh):
    ei0 = edge_index[0]
    ei1 = edge_index[1]
    offs = [0, EPARTS[0], E]
    epws = [ep // NW for ep in EPARTS]
    nchg = [w // CHG for w in epws]
    SCAP = 64
    srcg, dstg, srcm, dstm, msg_stages = [], [], [], [], []
    for p in range(P):
        sl = slice(offs[p], offs[p + 1])
        srcg.append(ei0[sl].reshape(NW, nchg[p], CHG))
        dstg.append(ei1[sl].reshape(NW, nchg[p], CHG))
        nch = epws[p] // CHM
        msg_stages.append([SCAP, nch - SCAP])

        def stage4(a, sl=sl, p=p):
            a = a[sl].reshape(NW, epws[p])
            pad = 2 * SCAP * CHM - epws[p]
            if pad:
                a = jnp.pad(a, ((0, 0), (0, pad)))
            return a.reshape(NW, 2, SCAP, CHM)

        srcm.append(stage4(ei0))
        dstm.append(stage4(ei1))
    L = W1.shape[0]

    xp = jnp.pad(x, ((0, NP - N), (0, 0)))
    zeros_np = jnp.zeros((NP, H), jnp.float32)

    h = _matmul_bias(xp, Wn, bn, NB)
    e = [_matmul_bias(edge_attr[offs[p]:offs[p + 1]], We, be, EBLK)
         for p in range(P)]

    Whp = jnp.pad(Wh, ((0, 0), (0, H - Wh.shape[1])))
    bhp = jnp.pad(bh, (0, H - bh.shape[0])).reshape(1, H)

    pred = None
    for i in range(L):
        parts = [_msg_scatter(h, e[p], srcm[p], dstm[p], zeros_np,
                              epws[p], msg_stages[p], SCAP)
                 for p in range(P)]
        epsb = jnp.full((1, H), 1.0, jnp.float32) + eps[i]
        z, s, s2 = _node_z(h, parts[0][0], parts[0][1],
                           parts[1][0], parts[1][1],
                           W1[i], b1[i], W2[i], b2[i], epsb)
        if i < L - 1:
            h, A, B = _node_bn_ab(z, h, s, s2, gamma[i], beta[i],
                                  We1[i, :H], We1[i, H:2 * H])
            G = [_gather_ab(A, B, srcg[p], dstg[p], EPARTS[p],
                            epws[p], nchg[p]) for p in range(P)]
            e = [_edge_mlp(e[p], G[p], We1[i, 2 * H:], be1[i],
                           We2[i], be2[i]) for p in range(P)]
        else:
            pred = _node_bn_head(z, h, s, s2, gamma[i], beta[i], Whp, bhp)

    return pred[:N, :1]


# speedup vs baseline: 1.1503x; 1.1503x over previous
import functools

import jax
import jax.numpy as jnp
from jax import lax
from jax.experimental import pallas as pl
from jax.experimental.pallas import tpu as pltpu
from jax.experimental.pallas import tpu_sc as plsc

N = 10000
NP = 10240
E = 320000
H = 128
NB = 512
GN = NP // NB
EBLK = 1280
GE = E // EBLK

NC = 2
NS = 16
NW = NC * NS
P = 2
EPARTS = (163840, 156160)
CHM = 40
CHG = 80
MSTAGE = 2
RPT = NP // NS


def _matmul_bias_kernel(x_ref, w_ref, b_ref, o_ref):
    o_ref[...] = (
        jnp.dot(x_ref[...], w_ref[...], preferred_element_type=jnp.float32)
        + b_ref[...]
    )


def _matmul_bias(x, W, b, blk):
    M, K = x.shape
    Ho = W.shape[1]
    return pl.pallas_call(
        _matmul_bias_kernel,
        grid=(M // blk,),
        in_specs=[
            pl.BlockSpec((blk, K), lambda i: (i, 0)),
            pl.BlockSpec((K, Ho), lambda i: (0, 0)),
            pl.BlockSpec((1, Ho), lambda i: (0, 0)),
        ],
        out_specs=pl.BlockSpec((blk, Ho), lambda i: (i, 0)),
        out_shape=jax.ShapeDtypeStruct((M, Ho), jnp.float32),
    )(x, W, b.reshape(1, Ho))


def _node_z_kernel(h_ref, p0_ref, p1_ref, p2_ref, p3_ref,
                   w1_ref, b1_ref, w2_ref, b2_ref,
                   epsb_ref, z_ref, s_ref, s2_ref):
    i = pl.program_id(0)
    z = (epsb_ref[...] * h_ref[...] + (p0_ref[...] + p1_ref[...])
         + (p2_ref[...] + p3_ref[...]))
    z = jnp.maximum(
        jnp.dot(z, w1_ref[...], preferred_element_type=jnp.float32)
        + b1_ref[...], 0.0)
    z = jnp.dot(z, w2_ref[...], preferred_element_type=jnp.float32) + b2_ref[...]
    z_ref[...] = z
    rows = i * NB + lax.broadcasted_iota(jnp.int32, (NB, 1), 0)
    zm = jnp.where(rows < N, z, 0.0)
    s_ref[...] = jnp.sum(zm, axis=0, keepdims=True)[None]
    s2_ref[...] = jnp.sum(zm * zm, axis=0, keepdims=True)[None]


def _node_z(h, p0, p1, p2, p3, W1l, b1l, W2l, b2l, epsb):
    return pl.pallas_call(
        _node_z_kernel,
        grid=(GN,),
        in_specs=[
            pl.BlockSpec((NB, H), lambda i: (i, 0)),
            pl.BlockSpec((NB, H), lambda i: (i, 0)),
            pl.BlockSpec((NB, H), lambda i: (i, 0)),
            pl.BlockSpec((NB, H), lambda i: (i, 0)),
            pl.BlockSpec((NB, H), lambda i: (i, 0)),
            pl.BlockSpec((H, H), lambda i: (0, 0)),
            pl.BlockSpec((1, H), lambda i: (0, 0)),
            pl.BlockSpec((H, H), lambda i: (0, 0)),
            pl.BlockSpec((1, H), lambda i: (0, 0)),
            pl.BlockSpec((1, H), lambda i: (0, 0)),
        ],
        out_specs=[
            pl.BlockSpec((NB, H), lambda i: (i, 0)),
            pl.BlockSpec((1, 1, H), lambda i: (i, 0, 0)),
            pl.BlockSpec((1, 1, H), lambda i: (i, 0, 0)),
        ],
        out_shape=[
            jax.ShapeDtypeStruct((NP, H), jnp.float32),
            jax.ShapeDtypeStruct((GN, 1, H), jnp.float32),
            jax.ShapeDtypeStruct((GN, 1, H), jnp.float32),
        ],
    )(h, p0, p1, p2, p3, W1l, b1l.reshape(1, H), W2l, b2l.reshape(1, H), epsb)


def _bn_update(z_ref, h_ref, s_ref, s2_ref, g_ref, be_ref):
    S = jnp.sum(s_ref[...], axis=0)
    S2 = jnp.sum(s2_ref[...], axis=0)
    mu = S * (1.0 / N)
    var = S2 * (1.0 / N) - mu * mu
    inv = lax.rsqrt(var + 1e-5)
    zbn = g_ref[...] * (z_ref[...] - mu) * inv + be_ref[...]
    return (h_ref[...] + jnp.maximum(zbn, 0.0)) * 0.5


def _node_bn_ab_kernel(z_ref, h_ref, s_ref, s2_ref, g_ref, be_ref,
                       wa_ref, wb_ref, h_out, a_out, b_out):
    hn = _bn_update(z_ref, h_ref, s_ref, s2_ref, g_ref, be_ref)
    h_out[...] = hn
    a_out[...] = jnp.dot(hn, wa_ref[...], preferred_element_type=jnp.float32)
    b_out[...] = jnp.dot(hn, wb_ref[...], preferred_element_type=jnp.float32)


def _node_bn_ab(z, h, s, s2, gl, bl, Wa, Wb):
    return pl.pallas_call(
        _node_bn_ab_kernel,
        grid=(GN,),
        in_specs=[
            pl.BlockSpec((NB, H), lambda i: (i, 0)),
            pl.BlockSpec((NB, H), lambda i: (i, 0)),
            pl.BlockSpec((GN, 1, H), lambda i: (0, 0, 0)),
            pl.BlockSpec((GN, 1, H), lambda i: (0, 0, 0)),
            pl.BlockSpec((1, H), lambda i: (0, 0)),
            pl.BlockSpec((1, H), lambda i: (0, 0)),
            pl.BlockSpec((H, H), lambda i: (0, 0)),
            pl.BlockSpec((H, H), lambda i: (0, 0)),
        ],
        out_specs=[
            pl.BlockSpec((NB, H), lambda i: (i, 0)),
            pl.BlockSpec((NB, H), lambda i: (i, 0)),
            pl.BlockSpec((NB, H), lambda i: (i, 0)),
        ],
        out_shape=[
            jax.ShapeDtypeStruct((NP, H), jnp.float32),
            jax.ShapeDtypeStruct((NP, H), jnp.float32),
            jax.ShapeDtypeStruct((NP, H), jnp.float32),
        ],
    )(z, h, s, s2, gl.reshape(1, H), bl.reshape(1, H), Wa, Wb)


def _node_bn_head_kernel(z_ref, h_ref, s_ref, s2_ref, g_ref, be_ref,
                         wh_ref, bh_ref, p_out):
    hn = _bn_update(z_ref, h_ref, s_ref, s2_ref, g_ref, be_ref)
    p_out[...] = (
        jnp.dot(hn, wh_ref[...], preferred_element_type=jnp.float32)
        + bh_ref[...]
    )


def _node_bn_head(z, h, s, s2, gl, bl, Whp, bhp):
    return pl.pallas_call(
        _node_bn_head_kernel,
        grid=(GN,),
        in_specs=[
            pl.BlockSpec((NB, H), lambda i: (i, 0)),
            pl.BlockSpec((NB, H), lambda i: (i, 0)),
            pl.BlockSpec((GN, 1, H), lambda i: (0, 0, 0)),
            pl.BlockSpec((GN, 1, H), lambda i: (0, 0, 0)),
            pl.BlockSpec((1, H), lambda i: (0, 0)),
            pl.BlockSpec((1, H), lambda i: (0, 0)),
            pl.BlockSpec((H, H), lambda i: (0, 0)),
            pl.BlockSpec((1, H), lambda i: (0, 0)),
        ],
        out_specs=pl.BlockSpec((NB, H), lambda i: (i, 0)),
        out_shape=jax.ShapeDtypeStruct((NP, H), jnp.float32),
    )(z, h, s, s2, gl.reshape(1, H), bl.reshape(1, H), Whp, bhp)


def _edge_mlp_kernel(e_ref, g_ref, we_ref, b1_ref, w2_ref, b2_ref, o_ref):
    e = e_ref[...]
    hid = jnp.maximum(
        jnp.dot(e, we_ref[...], preferred_element_type=jnp.float32)
        + g_ref[...] + b1_ref[...], 0.0)
    upd = jnp.dot(hid, w2_ref[...], preferred_element_type=jnp.float32) + b2_ref[...]
    o_ref[...] = e + 0.5 * upd


def _edge_mlp(e, G, We1e, be1l, We2l, be2l):
    ep = e.shape[0]
    return pl.pallas_call(
        _edge_mlp_kernel,
        grid=(ep // EBLK,),
        in_specs=[
            pl.BlockSpec((EBLK, H), lambda i: (i, 0)),
            pl.BlockSpec((EBLK, H), lambda i: (i, 0)),
            pl.BlockSpec((H, H), lambda i: (0, 0)),
            pl.BlockSpec((1, H), lambda i: (0, 0)),
            pl.BlockSpec((H, H), lambda i: (0, 0)),
            pl.BlockSpec((1, H), lambda i: (0, 0)),
        ],
        out_specs=pl.BlockSpec((EBLK, H), lambda i: (i, 0)),
        out_shape=jax.ShapeDtypeStruct((ep, H), jnp.float32),
    )(e, G, We1e, be1l.reshape(1, H), We2l, be2l.reshape(1, H))


def _sc_mesh():
    return plsc.VectorSubcoreMesh(core_axis_name="c", subcore_axis_name="s")


def _relu_add_rows(rows_v, ev_v, n_rows):
    def rbody(g, c2):
        r = 2 * g
        for rr in range(2):
            for cc in range(H // 16):
                sl = pl.ds(cc * 16, 16)
                rows_v[r + rr, sl] = jnp.maximum(
                    rows_v[r + rr, sl] + ev_v[r + rr, sl], 0.0)
        return c2

    lax.fori_loop(0, n_rows // 2, rbody, 0)


def _add_rows(ra_v, rb_v, n_rows):
    def rbody(g, c2):
        r = 2 * g
        for rr in range(2):
            for cc in range(H // 16):
                sl = pl.ds(cc * 16, 16)
                ra_v[r + rr, sl] = ra_v[r + rr, sl] + rb_v[r + rr, sl]
        return c2

    lax.fori_loop(0, n_rows // 2, rbody, 0)


def _msg_scatter(h, e, src4, dst4, zeros, epw, stage_counts, stage_cap):
    n_stages = len(stage_counts)

    @functools.partial(
        pl.kernel,
        mesh=_sc_mesh(),
        out_type=jax.ShapeDtypeStruct((NC, NP, H), jnp.float32),
        scratch_types=[
            pltpu.VMEM((stage_cap, CHM), jnp.int32),
            pltpu.VMEM((stage_cap, CHM), jnp.int32),
            pltpu.VMEM((CHM, H), jnp.float32),
            pltpu.VMEM((CHM, H), jnp.float32),
            pltpu.VMEM((CHM, H), jnp.float32),
            pltpu.VMEM((CHM, H), jnp.float32),
            pltpu.VMEM_SHARED((NP, H), jnp.float32),
            pltpu.SemaphoreType.DMA,
            pltpu.SemaphoreType.DMA,
            pltpu.SemaphoreType.DMA,
            pltpu.SemaphoreType.DMA,
        ],
    )
    def k(h_hbm, e_hbm, src_hbm, dst_hbm, z_hbm, out_hbm,
          srcb, dstb, rows0, rows1, ev0, ev1, acc, sg0, sg1, se0, se1):
        c = lax.axis_index("c")
        s = lax.axis_index("s")
        wid = s * NC + c
        base = wid * epw
        rows = (rows0, rows1)
        ev = (ev0, ev1)
        sg = (sg0, sg1)
        se = (se0, se1)

        pltpu.sync_copy(z_hbm.at[pl.ds(s * RPT, RPT)],
                        acc.at[pl.ds(s * RPT, RPT)])
        plsc.subcore_barrier()

        for part in range(n_stages):
            M = stage_counts[part]
            hoff = sum(stage_counts[:part])
            pltpu.sync_copy(src_hbm.at[wid, part], srcb)
            pltpu.sync_copy(dst_hbm.at[wid, part], dstb)

            def start(j, b):
                pltpu.async_copy(h_hbm.at[srcb.at[j]], rows[b], sg[b])
                pltpu.async_copy(
                    e_hbm.at[pl.ds(base + (hoff + j) * CHM, CHM)],
                    ev[b], se[b])

            def finish(j, b):
                pltpu.make_async_copy(h_hbm.at[srcb.at[j]], rows[b],
                                      sg[b]).wait()
                pltpu.make_async_copy(
                    e_hbm.at[pl.ds(base + (hoff + j) * CHM, CHM)],
                    ev[b], se[b]).wait()

            def step(j, b):
                finish(j, b)
                _relu_add_rows(rows[b], ev[b], CHM)
                pltpu.sync_copy(rows[b], acc.at[dstb.at[j]], add=True)
                jn = j + 2

                @pl.when(jn < M)
                def _():
                    start(jn, b)

            start(0, 0)
            start(1, 1)

            def body(g, carry):
                step(2 * g, 0)
                step(2 * g + 1, 1)
                return carry

            lax.fori_loop(0, M // 2, body, 0)
            if M % 2:
                step(M - 1, 0)

        plsc.subcore_barrier()
        pltpu.sync_copy(acc.at[pl.ds(s * RPT, RPT)],
                        out_hbm.at[c, pl.ds(s * RPT, RPT)])

    return k(h, e, src4, dst4, zeros)


def _gather_ab(A, B, src3, dst3, ep, epw, nch):
    @functools.partial(
        pl.kernel,
        mesh=_sc_mesh(),
        out_type=jax.ShapeDtypeStruct((ep, H), jnp.float32),
        scratch_types=[
            pltpu.VMEM((nch, CHG), jnp.int32),
            pltpu.VMEM((nch, CHG), jnp.int32),
            pltpu.VMEM((CHG, H), jnp.float32),
            pltpu.VMEM((CHG, H), jnp.float32),
            pltpu.VMEM((CHG, H), jnp.float32),
            pltpu.VMEM((CHG, H), jnp.float32),
            pltpu.SemaphoreType.DMA,
            pltpu.SemaphoreType.DMA,
            pltpu.SemaphoreType.DMA,
            pltpu.SemaphoreType.DMA,
        ],
    )
    def k(a_hbm, b_hbm, src_hbm, dst_hbm, out_hbm,
          srcb, dstb, ra0, ra1, rb0, rb1, sa0, sa1, sb0, sb1):
        c = lax.axis_index("c")
        s = lax.axis_index("s")
        wid = s * NC + c
        base = wid * epw
        ra = (ra0, ra1)
        rb = (rb0, rb1)
        sa = (sa0, sa1)
        sb = (sb0, sb1)

        pltpu.sync_copy(src_hbm.at[wid], srcb)
        pltpu.sync_copy(dst_hbm.at[wid], dstb)

        def start(j, b):
            pltpu.async_copy(a_hbm.at[srcb.at[j]], ra[b], sa[b])
            pltpu.async_copy(b_hbm.at[dstb.at[j]], rb[b], sb[b])

        def finish(j, b):
            pltpu.make_async_copy(a_hbm.at[srcb.at[j]], ra[b], sa[b]).wait()
            pltpu.make_async_copy(b_hbm.at[dstb.at[j]], rb[b], sb[b]).wait()

        def step(j, b):
            finish(j, b)
            _add_rows(ra[b], rb[b], CHG)
            pltpu.sync_copy(ra[b], out_hbm.at[pl.ds(base + j * CHG, CHG)])
            jn = j + 2

            @pl.when(jn < nch)
            def _():
                start(jn, b)

        start(0, 0)
        start(1, 1)

        def body(g, carry):
            step(2 * g, 0)
            step(2 * g + 1, 1)
            return carry

        lax.fori_loop(0, nch // 2, body, 0)
        if nch % 2:
            step(nch - 1, 0)

    return k(A, B, src3, dst3)


def kernel(x, edge_index, edge_attr, Wn, bn, We, be, eps, W1, b1, W2, b2,
           gamma, beta, We1, be1, We2, be2, Wh, bh):
    ei0 = edge_index[0]
    ei1 = edge_index[1]
    offs = [0, EPARTS[0], E]
    epws = [ep // NW for ep in EPARTS]
    nchg = [w // CHG for w in epws]
    SCAP = 64
    srcg, dstg, srcm, dstm, msg_stages = [], [], [], [], []
    for p in range(P):
        sl = slice(offs[p], offs[p + 1])
        srcg.append(ei0[sl].reshape(NW, nchg[p], CHG))
        dstg.append(ei1[sl].reshape(NW, nchg[p], CHG))
        nch = epws[p] // CHM
        msg_stages.append([SCAP, nch - SCAP])

        def stage4(a, sl=sl, p=p):
            a = a[sl].reshape(NW, epws[p])
            pad = 2 * SCAP * CHM - epws[p]
            if pad:
                a = jnp.pad(a, ((0, 0), (0, pad)))
            return a.reshape(NW, 2, SCAP, CHM)

        srcm.append(stage4(ei0))
        dstm.append(stage4(ei1))
    L = W1.shape[0]

    xp = jnp.pad(x, ((0, NP - N), (0, 0)))
    zeros_np = jnp.zeros((NP, H), jnp.float32)

    h = _matmul_bias(xp, Wn, bn, NB)
    e = [_matmul_bias(edge_attr[offs[p]:offs[p + 1]], We, be, EBLK)
         for p in range(P)]

    Whp = jnp.pad(Wh, ((0, 0), (0, H - Wh.shape[1])))
    bhp = jnp.pad(bh, (0, H - bh.shape[0])).reshape(1, H)

    pred = None
    for i in range(L):
        parts = [_msg_scatter(h, e[p], srcm[p], dstm[p], zeros_np,
                              epws[p], msg_stages[p], SCAP)
                 for p in range(P)]
        epsb = jnp.full((1, H), 1.0, jnp.float32) + eps[i]
        z, s, s2 = _node_z(h, parts[0][0], parts[0][1],
                           parts[1][0], parts[1][1],
                           W1[i], b1[i], W2[i], b2[i], epsb)
        if i < L - 1:
            h, A, B = _node_bn_ab(z, h, s, s2, gamma[i], beta[i],
                                  We1[i, :H], We1[i, H:2 * H])
            G = [_gather_ab(A, B, srcg[p], dstg[p], EPARTS[p],
                            epws[p], nchg[p]) for p in range(P)]
            e = [_edge_mlp(e[p], G[p], We1[i, 2 * H:], be1[i],
                           We2[i], be2[i]) for p in range(P)]
        else:
            pred = _node_bn_head(z, h, s, s2, gamma[i], beta[i], Whp, bhp)

    return pred[:N, :1]


# speedup vs baseline: 1.2284x; 1.0679x over previous
import functools

import jax
import jax.numpy as jnp
from jax import lax
from jax.experimental import pallas as pl
from jax.experimental.pallas import tpu as pltpu
from jax.experimental.pallas import tpu_sc as plsc

N = 10000
NP = 10240
E = 320000
H = 128
NB = 512
GN = NP // NB
EBLK = 2560

NC = 2
NS = 16
NW = NC * NS
P = 2
EPARTS = (163840, 156160)
CHM = 40
CHG = 80
MSTAGE = 2
RPT = NP // NS


def _matmul_bias_kernel(x_ref, w_ref, b_ref, o_ref):
    o_ref[...] = (
        jnp.dot(x_ref[...], w_ref[...], preferred_element_type=jnp.float32)
        + b_ref[...]
    )


def _matmul_bias(x, W, b, blk):
    M, K = x.shape
    Ho = W.shape[1]
    return pl.pallas_call(
        _matmul_bias_kernel,
        grid=(M // blk,),
        in_specs=[
            pl.BlockSpec((blk, K), lambda i: (i, 0)),
            pl.BlockSpec((K, Ho), lambda i: (0, 0)),
            pl.BlockSpec((1, Ho), lambda i: (0, 0)),
        ],
        out_specs=pl.BlockSpec((blk, Ho), lambda i: (i, 0)),
        out_shape=jax.ShapeDtypeStruct((M, Ho), jnp.float32),
    )(x, W, b.reshape(1, Ho))


def _node_z_kernel(h_ref, p0_ref, p1_ref, p2_ref, p3_ref,
                   w1_ref, b1_ref, w2_ref, b2_ref,
                   epsb_ref, z_ref, s_ref, s2_ref):
    i = pl.program_id(0)
    z = (epsb_ref[...] * h_ref[...] + (p0_ref[...] + p1_ref[...])
         + (p2_ref[...] + p3_ref[...]))
    z = jnp.maximum(
        jnp.dot(z, w1_ref[...], preferred_element_type=jnp.float32)
        + b1_ref[...], 0.0)
    z = jnp.dot(z, w2_ref[...], preferred_element_type=jnp.float32) + b2_ref[...]
    z_ref[...] = z
    rows = i * NB + lax.broadcasted_iota(jnp.int32, (NB, 1), 0)
    zm = jnp.where(rows < N, z, 0.0)
    s_ref[...] = jnp.sum(zm, axis=0, keepdims=True)[None]
    s2_ref[...] = jnp.sum(zm * zm, axis=0, keepdims=True)[None]


def _node_z(h, p0, p1, p2, p3, W1l, b1l, W2l, b2l, epsb):
    return pl.pallas_call(
        _node_z_kernel,
        grid=(GN,),
        in_specs=[
            pl.BlockSpec((NB, H), lambda i: (i, 0)),
            pl.BlockSpec((NB, H), lambda i: (i, 0)),
            pl.BlockSpec((NB, H), lambda i: (i, 0)),
            pl.BlockSpec((NB, H), lambda i: (i, 0)),
            pl.BlockSpec((NB, H), lambda i: (i, 0)),
            pl.BlockSpec((H, H), lambda i: (0, 0)),
            pl.BlockSpec((1, H), lambda i: (0, 0)),
            pl.BlockSpec((H, H), lambda i: (0, 0)),
            pl.BlockSpec((1, H), lambda i: (0, 0)),
            pl.BlockSpec((1, H), lambda i: (0, 0)),
        ],
        out_specs=[
            pl.BlockSpec((NB, H), lambda i: (i, 0)),
            pl.BlockSpec((1, 1, H), lambda i: (i, 0, 0)),
            pl.BlockSpec((1, 1, H), lambda i: (i, 0, 0)),
        ],
        out_shape=[
            jax.ShapeDtypeStruct((NP, H), jnp.float32),
            jax.ShapeDtypeStruct((GN, 1, H), jnp.float32),
            jax.ShapeDtypeStruct((GN, 1, H), jnp.float32),
        ],
    )(h, p0, p1, p2, p3, W1l, b1l.reshape(1, H), W2l, b2l.reshape(1, H), epsb)


def _bn_update(z_ref, h_ref, s_ref, s2_ref, g_ref, be_ref):
    S = jnp.sum(s_ref[...], axis=0)
    S2 = jnp.sum(s2_ref[...], axis=0)
    mu = S * (1.0 / N)
    var = S2 * (1.0 / N) - mu * mu
    inv = lax.rsqrt(var + 1e-5)
    zbn = g_ref[...] * (z_ref[...] - mu) * inv + be_ref[...]
    return (h_ref[...] + jnp.maximum(zbn, 0.0)) * 0.5


def _node_bn_ab_kernel(z_ref, h_ref, s_ref, s2_ref, g_ref, be_ref,
                       wa_ref, wb_ref, h_out, a_out, b_out):
    hn = _bn_update(z_ref, h_ref, s_ref, s2_ref, g_ref, be_ref)
    h_out[...] = hn
    a_out[...] = jnp.dot(hn, wa_ref[...], preferred_element_type=jnp.float32)
    b_out[...] = jnp.dot(hn, wb_ref[...], preferred_element_type=jnp.float32)


def _node_bn_ab(z, h, s, s2, gl, bl, Wa, Wb):
    return pl.pallas_call(
        _node_bn_ab_kernel,
        grid=(GN,),
        in_specs=[
            pl.BlockSpec((NB, H), lambda i: (i, 0)),
            pl.BlockSpec((NB, H), lambda i: (i, 0)),
            pl.BlockSpec((GN, 1, H), lambda i: (0, 0, 0)),
            pl.BlockSpec((GN, 1, H), lambda i: (0, 0, 0)),
            pl.BlockSpec((1, H), lambda i: (0, 0)),
            pl.BlockSpec((1, H), lambda i: (0, 0)),
            pl.BlockSpec((H, H), lambda i: (0, 0)),
            pl.BlockSpec((H, H), lambda i: (0, 0)),
        ],
        out_specs=[
            pl.BlockSpec((NB, H), lambda i: (i, 0)),
            pl.BlockSpec((NB, H), lambda i: (i, 0)),
            pl.BlockSpec((NB, H), lambda i: (i, 0)),
        ],
        out_shape=[
            jax.ShapeDtypeStruct((NP, H), jnp.float32),
            jax.ShapeDtypeStruct((NP, H), jnp.float32),
            jax.ShapeDtypeStruct((NP, H), jnp.float32),
        ],
    )(z, h, s, s2, gl.reshape(1, H), bl.reshape(1, H), Wa, Wb)


def _node_bn_head_kernel(z_ref, h_ref, s_ref, s2_ref, g_ref, be_ref,
                         wh_ref, bh_ref, p_out):
    hn = _bn_update(z_ref, h_ref, s_ref, s2_ref, g_ref, be_ref)
    p_out[...] = (
        jnp.dot(hn, wh_ref[...], preferred_element_type=jnp.float32)
        + bh_ref[...]
    )


def _node_bn_head(z, h, s, s2, gl, bl, Whp, bhp):
    return pl.pallas_call(
        _node_bn_head_kernel,
        grid=(GN,),
        in_specs=[
            pl.BlockSpec((NB, H), lambda i: (i, 0)),
            pl.BlockSpec((NB, H), lambda i: (i, 0)),
            pl.BlockSpec((GN, 1, H), lambda i: (0, 0, 0)),
            pl.BlockSpec((GN, 1, H), lambda i: (0, 0, 0)),
            pl.BlockSpec((1, H), lambda i: (0, 0)),
            pl.BlockSpec((1, H), lambda i: (0, 0)),
            pl.BlockSpec((H, H), lambda i: (0, 0)),
            pl.BlockSpec((1, H), lambda i: (0, 0)),
        ],
        out_specs=pl.BlockSpec((NB, H), lambda i: (i, 0)),
        out_shape=jax.ShapeDtypeStruct((NP, H), jnp.float32),
    )(z, h, s, s2, gl.reshape(1, H), bl.reshape(1, H), Whp, bhp)


def _edge_mlp_kernel(e_ref, g_ref, we_ref, b1_ref, w2_ref, b2_ref, o_ref):
    e = e_ref[...]
    hid = jnp.maximum(
        jnp.dot(e, we_ref[...], preferred_element_type=jnp.float32)
        + g_ref[...] + b1_ref[...], 0.0)
    upd = jnp.dot(hid, w2_ref[...], preferred_element_type=jnp.float32) + b2_ref[...]
    o_ref[...] = e + 0.5 * upd


def _edge_mlp(e, G, We1e, be1l, We2l, be2l):
    ep = e.shape[0]
    return pl.pallas_call(
        _edge_mlp_kernel,
        grid=(ep // EBLK,),
        in_specs=[
            pl.BlockSpec((EBLK, H), lambda i: (i, 0)),
            pl.BlockSpec((EBLK, H), lambda i: (i, 0)),
            pl.BlockSpec((H, H), lambda i: (0, 0)),
            pl.BlockSpec((1, H), lambda i: (0, 0)),
            pl.BlockSpec((H, H), lambda i: (0, 0)),
            pl.BlockSpec((1, H), lambda i: (0, 0)),
        ],
        out_specs=pl.BlockSpec((EBLK, H), lambda i: (i, 0)),
        out_shape=jax.ShapeDtypeStruct((ep, H), jnp.float32),
    )(e, G, We1e, be1l.reshape(1, H), We2l, be2l.reshape(1, H))


def _sc_mesh():
    return plsc.VectorSubcoreMesh(core_axis_name="c", subcore_axis_name="s")


def _relu_add_rows(rows_v, ev_v, n_rows):
    def rbody(g, c2):
        r = 2 * g
        for rr in range(2):
            for cc in range(H // 16):
                sl = pl.ds(cc * 16, 16)
                rows_v[r + rr, sl] = jnp.maximum(
                    rows_v[r + rr, sl] + ev_v[r + rr, sl], 0.0)
        return c2

    lax.fori_loop(0, n_rows // 2, rbody, 0)


def _add_rows(ra_v, rb_v, n_rows):
    def rbody(g, c2):
        r = 2 * g
        for rr in range(2):
            for cc in range(H // 16):
                sl = pl.ds(cc * 16, 16)
                ra_v[r + rr, sl] = ra_v[r + rr, sl] + rb_v[r + rr, sl]
        return c2

    lax.fori_loop(0, n_rows // 2, rbody, 0)


def _msg_scatter(h, e, src4, dst4, zeros, epw, stage_counts, stage_cap):
    n_stages = len(stage_counts)

    @functools.partial(
        pl.kernel,
        mesh=_sc_mesh(),
        out_type=jax.ShapeDtypeStruct((NC, NP, H), jnp.float32),
        scratch_types=[
            pltpu.VMEM((stage_cap, CHM), jnp.int32),
            pltpu.VMEM((stage_cap, CHM), jnp.int32),
            pltpu.VMEM((CHM, H), jnp.float32),
            pltpu.VMEM((CHM, H), jnp.float32),
            pltpu.VMEM((CHM, H), jnp.float32),
            pltpu.VMEM((CHM, H), jnp.float32),
            pltpu.VMEM_SHARED((NP, H), jnp.float32),
            pltpu.SemaphoreType.DMA,
            pltpu.SemaphoreType.DMA,
            pltpu.SemaphoreType.DMA,
            pltpu.SemaphoreType.DMA,
        ],
    )
    def k(h_hbm, e_hbm, src_hbm, dst_hbm, z_hbm, out_hbm,
          srcb, dstb, rows0, rows1, ev0, ev1, acc, sg0, sg1, se0, se1):
        c = lax.axis_index("c")
        s = lax.axis_index("s")
        wid = s * NC + c
        base = wid * epw
        rows = (rows0, rows1)
        ev = (ev0, ev1)
        sg = (sg0, sg1)
        se = (se0, se1)

        pltpu.sync_copy(z_hbm.at[pl.ds(s * RPT, RPT)],
                        acc.at[pl.ds(s * RPT, RPT)])
        plsc.subcore_barrier()

        for part in range(n_stages):
            M = stage_counts[part]
            hoff = sum(stage_counts[:part])
            pltpu.sync_copy(src_hbm.at[wid, part], srcb)
            pltpu.sync_copy(dst_hbm.at[wid, part], dstb)

            def start(j, b):
                pltpu.async_copy(h_hbm.at[srcb.at[j]], rows[b], sg[b])
                pltpu.async_copy(
                    e_hbm.at[pl.ds(base + (hoff + j) * CHM, CHM)],
                    ev[b], se[b])

            def finish(j, b):
                pltpu.make_async_copy(h_hbm.at[srcb.at[j]], rows[b],
                                      sg[b]).wait()
                pltpu.make_async_copy(
                    e_hbm.at[pl.ds(base + (hoff + j) * CHM, CHM)],
                    ev[b], se[b]).wait()

            def step(j, b):
                finish(j, b)
                _relu_add_rows(rows[b], ev[b], CHM)
                pltpu.sync_copy(rows[b], acc.at[dstb.at[j]], add=True)
                jn = j + 2

                @pl.when(jn < M)
                def _():
                    start(jn, b)

            start(0, 0)
            start(1, 1)

            def body(g, carry):
                step(2 * g, 0)
                step(2 * g + 1, 1)
                return carry

            lax.fori_loop(0, M // 2, body, 0)
            if M % 2:
                step(M - 1, 0)

        plsc.subcore_barrier()
        pltpu.sync_copy(acc.at[pl.ds(s * RPT, RPT)],
                        out_hbm.at[c, pl.ds(s * RPT, RPT)])

    return k(h, e, src4, dst4, zeros)


def _gather_ab(A, B, src3, dst3, ep, epw, nch):
    @functools.partial(
        pl.kernel,
        mesh=_sc_mesh(),
        out_type=jax.ShapeDtypeStruct((ep, H), jnp.float32),
        scratch_types=[
            pltpu.VMEM((nch, CHG), jnp.int32),
            pltpu.VMEM((nch, CHG), jnp.int32),
            pltpu.VMEM((CHG, H), jnp.float32),
            pltpu.VMEM((CHG, H), jnp.float32),
            pltpu.VMEM((CHG, H), jnp.float32),
            pltpu.VMEM((CHG, H), jnp.float32),
            pltpu.VMEM((CHG, H), jnp.float32),
            pltpu.VMEM((CHG, H), jnp.float32),
            pltpu.SemaphoreType.DMA,
            pltpu.SemaphoreType.DMA,
            pltpu.SemaphoreType.DMA,
            pltpu.SemaphoreType.DMA,
            pltpu.SemaphoreType.DMA,
            pltpu.SemaphoreType.DMA,
            pltpu.SemaphoreType.DMA,
            pltpu.SemaphoreType.DMA,
            pltpu.SemaphoreType.DMA,
        ],
    )
    def k(a_hbm, b_hbm, src_hbm, dst_hbm, out_hbm,
          srcb, dstb, ra0, ra1, ra2, rb0, rb1, rb2,
          sa0, sa1, sa2, sb0, sb1, sb2, so0, so1, so2):
        c = lax.axis_index("c")
        s = lax.axis_index("s")
        wid = s * NC + c
        base = wid * epw
        ra = (ra0, ra1, ra2)
        rb = (rb0, rb1, rb2)
        sa = (sa0, sa1, sa2)
        sb = (sb0, sb1, sb2)
        so = (so0, so1, so2)

        pltpu.sync_copy(src_hbm.at[wid], srcb)
        pltpu.sync_copy(dst_hbm.at[wid], dstb)

        def start(j, b):
            pltpu.async_copy(a_hbm.at[srcb.at[j]], ra[b], sa[b])
            pltpu.async_copy(b_hbm.at[dstb.at[j]], rb[b], sb[b])

        def finish(j, b):
            pltpu.make_async_copy(a_hbm.at[srcb.at[j]], ra[b], sa[b]).wait()
            pltpu.make_async_copy(b_hbm.at[dstb.at[j]], rb[b], sb[b]).wait()

        def st_start(j, b):
            pltpu.async_copy(ra[b], out_hbm.at[pl.ds(base + j * CHG, CHG)],
                             so[b])

        def st_wait(j, b):
            pltpu.make_async_copy(ra[b],
                                  out_hbm.at[pl.ds(base + j * CHG, CHG)],
                                  so[b]).wait()

        def step(j, b):
            finish(j, b)
            _add_rows(ra[b], rb[b], CHG)
            st_start(j, b)
            bn = (b + 2) % 3

            @pl.when((j + 2 < nch) & (j >= 1))
            def _():
                st_wait(j - 1, bn)

            @pl.when(j + 2 < nch)
            def _():
                start(j + 2, bn)

        start(0, 0)
        start(1, 1)

        def body(g, carry):
            step(3 * g, 0)
            step(3 * g + 1, 1)
            step(3 * g + 2, 2)
            return carry

        lax.fori_loop(0, nch // 3, body, 0)
        for j in range(3 * (nch // 3), nch):
            step(j, j % 3)
        for j in range(max(0, nch - 3), nch):
            st_wait(j, j % 3)

    return k(A, B, src3, dst3)


def kernel(x, edge_index, edge_attr, Wn, bn, We, be, eps, W1, b1, W2, b2,
           gamma, beta, We1, be1, We2, be2, Wh, bh):
    ei0 = edge_index[0]
    ei1 = edge_index[1]
    offs = [0, EPARTS[0], E]
    epws = [ep // NW for ep in EPARTS]
    nchg = [w // CHG for w in epws]
    SCAP = 64
    srcg, dstg, srcm, dstm, msg_stages = [], [], [], [], []
    for p in range(P):
        sl = slice(offs[p], offs[p + 1])
        srcg.append(ei0[sl].reshape(NW, nchg[p], CHG))
        dstg.append(ei1[sl].reshape(NW, nchg[p], CHG))
        nch = epws[p] // CHM
        msg_stages.append([SCAP, nch - SCAP])

        def stage4(a, sl=sl, p=p):
            a = a[sl].reshape(NW, epws[p])
            pad = 2 * SCAP * CHM - epws[p]
            if pad:
                a = jnp.pad(a, ((0, 0), (0, pad)))
            return a.reshape(NW, 2, SCAP, CHM)

        srcm.append(stage4(ei0))
        dstm.append(stage4(ei1))
    L = W1.shape[0]

    xp = jnp.pad(x, ((0, NP - N), (0, 0)))
    zeros_np = jnp.zeros((NP, H), jnp.float32)

    h = _matmul_bias(xp, Wn, bn, NB)
    e = [_matmul_bias(edge_attr[offs[p]:offs[p + 1]], We, be, EBLK)
         for p in range(P)]

    Whp = jnp.pad(Wh, ((0, 0), (0, H - Wh.shape[1])))
    bhp = jnp.pad(bh, (0, H - bh.shape[0])).reshape(1, H)

    pred = None
    for i in range(L):
        parts = [_msg_scatter(h, e[p], srcm[p], dstm[p], zeros_np,
                              epws[p], msg_stages[p], SCAP)
                 for p in range(P)]
        epsb = jnp.full((1, H), 1.0, jnp.float32) + eps[i]
        z, s, s2 = _node_z(h, parts[0][0], parts[0][1],
                           parts[1][0], parts[1][1],
                           W1[i], b1[i], W2[i], b2[i], epsb)
        if i < L - 1:
            h, A, B = _node_bn_ab(z, h, s, s2, gamma[i], beta[i],
                                  We1[i, :H], We1[i, H:2 * H])
            G = [_gather_ab(A, B, srcg[p], dstg[p], EPARTS[p],
                            epws[p], nchg[p]) for p in range(P)]
            e = [_edge_mlp(e[p], G[p], We1[i, 2 * H:], be1[i],
                           We2[i], be2[i]) for p in range(P)]
        else:
            pred = _node_bn_head(z, h, s, s2, gamma[i], beta[i], Whp, bhp)

    return pred[:N, :1]


# speedup vs baseline: 1.2490x; 1.0168x over previous
import functools

import jax
import jax.numpy as jnp
from jax import lax
from jax.experimental import pallas as pl
from jax.experimental.pallas import tpu as pltpu
from jax.experimental.pallas import tpu_sc as plsc

N = 10000
NP = 10240
E = 320000
H = 128
NB = 512
GN = NP // NB
EBLK = 2560

NC = 2
NS = 16
NW = NC * NS
P = 2
EPARTS = (163840, 156160)
CHM = 40
CHG = 80
MSTAGE = 2
RPT = NP // NS


def _matmul_bias_kernel(x_ref, w_ref, b_ref, o_ref):
    o_ref[...] = (
        jnp.dot(x_ref[...], w_ref[...], preferred_element_type=jnp.float32)
        + b_ref[...]
    )


def _matmul_bias(x, W, b, blk):
    M, K = x.shape
    Ho = W.shape[1]
    return pl.pallas_call(
        _matmul_bias_kernel,
        grid=(M // blk,),
        in_specs=[
            pl.BlockSpec((blk, K), lambda i: (i, 0)),
            pl.BlockSpec((K, Ho), lambda i: (0, 0)),
            pl.BlockSpec((1, Ho), lambda i: (0, 0)),
        ],
        out_specs=pl.BlockSpec((blk, Ho), lambda i: (i, 0)),
        out_shape=jax.ShapeDtypeStruct((M, Ho), jnp.float32),
    )(x, W, b.reshape(1, Ho))


def _node_z_kernel(h_ref, pa_ref, pb_ref,
                   w1_ref, b1_ref, w2_ref, b2_ref,
                   epsb_ref, z_ref, s_ref, s2_ref):
    i = pl.program_id(0)
    z = (epsb_ref[...] * h_ref[...] + (pa_ref[0] + pa_ref[1])
         + (pb_ref[0] + pb_ref[1]))
    z = jnp.maximum(
        jnp.dot(z, w1_ref[...], preferred_element_type=jnp.float32)
        + b1_ref[...], 0.0)
    z = jnp.dot(z, w2_ref[...], preferred_element_type=jnp.float32) + b2_ref[...]
    z_ref[...] = z
    rows = i * NB + lax.broadcasted_iota(jnp.int32, (NB, 1), 0)
    zm = jnp.where(rows < N, z, 0.0)
    s_ref[...] = jnp.sum(zm, axis=0, keepdims=True)[None]
    s2_ref[...] = jnp.sum(zm * zm, axis=0, keepdims=True)[None]


def _node_z(h, pa, pb, W1l, b1l, W2l, b2l, epsb):
    return pl.pallas_call(
        _node_z_kernel,
        grid=(GN,),
        in_specs=[
            pl.BlockSpec((NB, H), lambda i: (i, 0)),
            pl.BlockSpec((NC, NB, H), lambda i: (0, i, 0)),
            pl.BlockSpec((NC, NB, H), lambda i: (0, i, 0)),
            pl.BlockSpec((H, H), lambda i: (0, 0)),
            pl.BlockSpec((1, H), lambda i: (0, 0)),
            pl.BlockSpec((H, H), lambda i: (0, 0)),
            pl.BlockSpec((1, H), lambda i: (0, 0)),
            pl.BlockSpec((1, H), lambda i: (0, 0)),
        ],
        out_specs=[
            pl.BlockSpec((NB, H), lambda i: (i, 0)),
            pl.BlockSpec((1, 1, H), lambda i: (i, 0, 0)),
            pl.BlockSpec((1, 1, H), lambda i: (i, 0, 0)),
        ],
        out_shape=[
            jax.ShapeDtypeStruct((NP, H), jnp.float32),
            jax.ShapeDtypeStruct((GN, 1, H), jnp.float32),
            jax.ShapeDtypeStruct((GN, 1, H), jnp.float32),
        ],
    )(h, pa, pb, W1l, b1l.reshape(1, H), W2l, b2l.reshape(1, H), epsb)


def _bn_update(z_ref, h_ref, s_ref, s2_ref, g_ref, be_ref):
    S = jnp.sum(s_ref[...], axis=0)
    S2 = jnp.sum(s2_ref[...], axis=0)
    mu = S * (1.0 / N)
    var = S2 * (1.0 / N) - mu * mu
    inv = lax.rsqrt(var + 1e-5)
    zbn = g_ref[...] * (z_ref[...] - mu) * inv + be_ref[...]
    return (h_ref[...] + jnp.maximum(zbn, 0.0)) * 0.5


def _node_bn_ab_kernel(z_ref, h_ref, s_ref, s2_ref, g_ref, be_ref,
                       wa_ref, wb_ref, h_out, a_out, b_out):
    hn = _bn_update(z_ref, h_ref, s_ref, s2_ref, g_ref, be_ref)
    h_out[...] = hn
    a_out[...] = jnp.dot(hn, wa_ref[...], preferred_element_type=jnp.float32)
    b_out[...] = jnp.dot(hn, wb_ref[...], preferred_element_type=jnp.float32)


def _node_bn_ab(z, h, s, s2, gl, bl, Wa, Wb):
    return pl.pallas_call(
        _node_bn_ab_kernel,
        grid=(GN,),
        in_specs=[
            pl.BlockSpec((NB, H), lambda i: (i, 0)),
            pl.BlockSpec((NB, H), lambda i: (i, 0)),
            pl.BlockSpec((GN, 1, H), lambda i: (0, 0, 0)),
            pl.BlockSpec((GN, 1, H), lambda i: (0, 0, 0)),
            pl.BlockSpec((1, H), lambda i: (0, 0)),
            pl.BlockSpec((1, H), lambda i: (0, 0)),
            pl.BlockSpec((H, H), lambda i: (0, 0)),
            pl.BlockSpec((H, H), lambda i: (0, 0)),
        ],
        out_specs=[
            pl.BlockSpec((NB, H), lambda i: (i, 0)),
            pl.BlockSpec((NB, H), lambda i: (i, 0)),
            pl.BlockSpec((NB, H), lambda i: (i, 0)),
        ],
        out_shape=[
            jax.ShapeDtypeStruct((NP, H), jnp.float32),
            jax.ShapeDtypeStruct((NP, H), jnp.float32),
            jax.ShapeDtypeStruct((NP, H), jnp.float32),
        ],
    )(z, h, s, s2, gl.reshape(1, H), bl.reshape(1, H), Wa, Wb)


def _node_bn_head_kernel(z_ref, h_ref, s_ref, s2_ref, g_ref, be_ref,
                         wh_ref, bh_ref, p_out):
    hn = _bn_update(z_ref, h_ref, s_ref, s2_ref, g_ref, be_ref)
    p_out[...] = (
        jnp.dot(hn, wh_ref[...], preferred_element_type=jnp.float32)
        + bh_ref[...]
    )


def _node_bn_head(z, h, s, s2, gl, bl, Whp, bhp):
    return pl.pallas_call(
        _node_bn_head_kernel,
        grid=(GN,),
        in_specs=[
            pl.BlockSpec((NB, H), lambda i: (i, 0)),
            pl.BlockSpec((NB, H), lambda i: (i, 0)),
            pl.BlockSpec((GN, 1, H), lambda i: (0, 0, 0)),
            pl.BlockSpec((GN, 1, H), lambda i: (0, 0, 0)),
            pl.BlockSpec((1, H), lambda i: (0, 0)),
            pl.BlockSpec((1, H), lambda i: (0, 0)),
            pl.BlockSpec((H, H), lambda i: (0, 0)),
            pl.BlockSpec((1, H), lambda i: (0, 0)),
        ],
        out_specs=pl.BlockSpec((NB, H), lambda i: (i, 0)),
        out_shape=jax.ShapeDtypeStruct((NP, H), jnp.float32),
    )(z, h, s, s2, gl.reshape(1, H), bl.reshape(1, H), Whp, bhp)


def _edge_mlp_kernel(e_ref, g_ref, we_ref, b1_ref, w2_ref, b2_ref, o_ref):
    e = e_ref[...]
    hid = jnp.maximum(
        jnp.dot(e, we_ref[...], preferred_element_type=jnp.float32)
        + g_ref[...] + b1_ref[...], 0.0)
    upd = jnp.dot(hid, w2_ref[...], preferred_element_type=jnp.float32) + b2_ref[...]
    o_ref[...] = e + 0.5 * upd


def _edge_mlp(e, G, We1e, be1l, We2l, be2l):
    ep = e.shape[0]
    return pl.pallas_call(
        _edge_mlp_kernel,
        grid=(ep // EBLK,),
        in_specs=[
            pl.BlockSpec((EBLK, H), lambda i: (i, 0)),
            pl.BlockSpec((EBLK, H), lambda i: (i, 0)),
            pl.BlockSpec((H, H), lambda i: (0, 0)),
            pl.BlockSpec((1, H), lambda i: (0, 0)),
            pl.BlockSpec((H, H), lambda i: (0, 0)),
            pl.BlockSpec((1, H), lambda i: (0, 0)),
        ],
        out_specs=pl.BlockSpec((EBLK, H), lambda i: (i, 0)),
        out_shape=jax.ShapeDtypeStruct((ep, H), jnp.float32),
    )(e, G, We1e, be1l.reshape(1, H), We2l, be2l.reshape(1, H))


def _sc_mesh():
    return plsc.VectorSubcoreMesh(core_axis_name="c", subcore_axis_name="s")


def _relu_add_rows(rows_v, ev_v, n_rows):
    def rbody(g, c2):
        r = 2 * g
        for rr in range(2):
            for cc in range(H // 16):
                sl = pl.ds(cc * 16, 16)
                rows_v[r + rr, sl] = jnp.maximum(
                    rows_v[r + rr, sl] + ev_v[r + rr, sl], 0.0)
        return c2

    lax.fori_loop(0, n_rows // 2, rbody, 0)


def _add_rows(ra_v, rb_v, n_rows):
    def rbody(g, c2):
        r = 2 * g
        for rr in range(2):
            for cc in range(H // 16):
                sl = pl.ds(cc * 16, 16)
                ra_v[r + rr, sl] = ra_v[r + rr, sl] + rb_v[r + rr, sl]
        return c2

    lax.fori_loop(0, n_rows // 2, rbody, 0)


def _msg_scatter(h, e, src4, dst4, zeros, epw, stage_counts, stage_cap):
    n_stages = len(stage_counts)

    @functools.partial(
        pl.kernel,
        mesh=_sc_mesh(),
        out_type=jax.ShapeDtypeStruct((NC, NP, H), jnp.float32),
        scratch_types=[
            pltpu.VMEM((stage_cap, CHM), jnp.int32),
            pltpu.VMEM((stage_cap, CHM), jnp.int32),
            pltpu.VMEM((CHM, H), jnp.float32),
            pltpu.VMEM((CHM, H), jnp.float32),
            pltpu.VMEM((CHM, H), jnp.float32),
            pltpu.VMEM((CHM, H), jnp.float32),
            pltpu.VMEM_SHARED((NP, H), jnp.float32),
            pltpu.SemaphoreType.DMA,
            pltpu.SemaphoreType.DMA,
            pltpu.SemaphoreType.DMA,
            pltpu.SemaphoreType.DMA,
        ],
    )
    def k(h_hbm, e_hbm, src_hbm, dst_hbm, z_hbm, out_hbm,
          srcb, dstb, rows0, rows1, ev0, ev1, acc, sg0, sg1, se0, se1):
        c = lax.axis_index("c")
        s = lax.axis_index("s")
        wid = s * NC + c
        base = wid * epw
        rows = (rows0, rows1)
        ev = (ev0, ev1)
        sg = (sg0, sg1)
        se = (se0, se1)

        pltpu.sync_copy(z_hbm.at[pl.ds(s * RPT, RPT)],
                        acc.at[pl.ds(s * RPT, RPT)])
        plsc.subcore_barrier()

        for part in range(n_stages):
            M = stage_counts[part]
            hoff = sum(stage_counts[:part])
            pltpu.sync_copy(src_hbm.at[wid, part], srcb)
            pltpu.sync_copy(dst_hbm.at[wid, part], dstb)

            def start(j, b):
                pltpu.async_copy(h_hbm.at[srcb.at[j]], rows[b], sg[b])
                pltpu.async_copy(
                    e_hbm.at[pl.ds(base + (hoff + j) * CHM, CHM)],
                    ev[b], se[b])

            def finish(j, b):
                pltpu.make_async_copy(h_hbm.at[srcb.at[j]], rows[b],
                                      sg[b]).wait()
                pltpu.make_async_copy(
                    e_hbm.at[pl.ds(base + (hoff + j) * CHM, CHM)],
                    ev[b], se[b]).wait()

            def step(j, b):
                finish(j, b)
                _relu_add_rows(rows[b], ev[b], CHM)
                pltpu.sync_copy(rows[b], acc.at[dstb.at[j]], add=True)
                jn = j + 2

                @pl.when(jn < M)
                def _():
                    start(jn, b)

            start(0, 0)
            start(1, 1)

            def body(g, carry):
                step(2 * g, 0)
                step(2 * g + 1, 1)
                return carry

            lax.fori_loop(0, M // 2, body, 0)
            if M % 2:
                step(M - 1, 0)

        plsc.subcore_barrier()
        pltpu.sync_copy(acc.at[pl.ds(s * RPT, RPT)],
                        out_hbm.at[c, pl.ds(s * RPT, RPT)])

    return k(h, e, src4, dst4, zeros)


def _gather_ab(A, B, src3, dst3, ep, epw, nch):
    @functools.partial(
        pl.kernel,
        mesh=_sc_mesh(),
        out_type=jax.ShapeDtypeStruct((ep, H), jnp.float32),
        scratch_types=[
            pltpu.VMEM((nch, CHG), jnp.int32),
            pltpu.VMEM((nch, CHG), jnp.int32),
            pltpu.VMEM((CHG, H), jnp.float32),
            pltpu.VMEM((CHG, H), jnp.float32),
            pltpu.VMEM((CHG, H), jnp.float32),
            pltpu.VMEM((CHG, H), jnp.float32),
            pltpu.VMEM((CHG, H), jnp.float32),
            pltpu.VMEM((CHG, H), jnp.float32),
            pltpu.SemaphoreType.DMA,
            pltpu.SemaphoreType.DMA,
            pltpu.SemaphoreType.DMA,
            pltpu.SemaphoreType.DMA,
            pltpu.SemaphoreType.DMA,
            pltpu.SemaphoreType.DMA,
            pltpu.SemaphoreType.DMA,
            pltpu.SemaphoreType.DMA,
            pltpu.SemaphoreType.DMA,
        ],
    )
    def k(a_hbm, b_hbm, src_hbm, dst_hbm, out_hbm,
          srcb, dstb, ra0, ra1, ra2, rb0, rb1, rb2,
          sa0, sa1, sa2, sb0, sb1, sb2, so0, so1, so2):
        c = lax.axis_index("c")
        s = lax.axis_index("s")
        wid = s * NC + c
        base = wid * epw
        ra = (ra0, ra1, ra2)
        rb = (rb0, rb1, rb2)
        sa = (sa0, sa1, sa2)
        sb = (sb0, sb1, sb2)
        so = (so0, so1, so2)

        pltpu.sync_copy(src_hbm.at[wid], srcb)
        pltpu.sync_copy(dst_hbm.at[wid], dstb)

        def start(j, b):
            pltpu.async_copy(a_hbm.at[srcb.at[j]], ra[b], sa[b])
            pltpu.async_copy(b_hbm.at[dstb.at[j]], rb[b], sb[b])

        def finish(j, b):
            pltpu.make_async_copy(a_hbm.at[srcb.at[j]], ra[b], sa[b]).wait()
            pltpu.make_async_copy(b_hbm.at[dstb.at[j]], rb[b], sb[b]).wait()

        def st_start(j, b):
            pltpu.async_copy(ra[b], out_hbm.at[pl.ds(base + j * CHG, CHG)],
                             so[b])

        def st_wait(j, b):
            pltpu.make_async_copy(ra[b],
                                  out_hbm.at[pl.ds(base + j * CHG, CHG)],
                                  so[b]).wait()

        def step(j, b):
            finish(j, b)
            _add_rows(ra[b], rb[b], CHG)
            st_start(j, b)
            bn = (b + 2) % 3

            @pl.when((j + 2 < nch) & (j >= 1))
            def _():
                st_wait(j - 1, bn)

            @pl.when(j + 2 < nch)
            def _():
                start(j + 2, bn)

        start(0, 0)
        start(1, 1)

        def body(g, carry):
            step(3 * g, 0)
            step(3 * g + 1, 1)
            step(3 * g + 2, 2)
            return carry

        lax.fori_loop(0, nch // 3, body, 0)
        for j in range(3 * (nch // 3), nch):
            step(j, j % 3)
        for j in range(max(0, nch - 3), nch):
            st_wait(j, j % 3)

    return k(A, B, src3, dst3)


def kernel(x, edge_index, edge_attr, Wn, bn, We, be, eps, W1, b1, W2, b2,
           gamma, beta, We1, be1, We2, be2, Wh, bh):
    ei0 = edge_index[0]
    ei1 = edge_index[1]
    offs = [0, EPARTS[0], E]
    epws = [ep // NW for ep in EPARTS]
    nchg = [w // CHG for w in epws]
    SCAP = 64
    srcg, dstg, srcm, dstm, msg_stages = [], [], [], [], []
    for p in range(P):
        sl = slice(offs[p], offs[p + 1])
        srcg.append(ei0[sl].reshape(NW, nchg[p], CHG))
        dstg.append(ei1[sl].reshape(NW, nchg[p], CHG))
        nch = epws[p] // CHM
        msg_stages.append([SCAP, nch - SCAP])

        def stage4(a, sl=sl, p=p):
            a = a[sl].reshape(NW, epws[p])
            pad = 2 * SCAP * CHM - epws[p]
            if pad:
                a = jnp.pad(a, ((0, 0), (0, pad)))
            return a.reshape(NW, 2, SCAP, CHM)

        srcm.append(stage4(ei0))
        dstm.append(stage4(ei1))
    L = W1.shape[0]

    xp = jnp.pad(x, ((0, NP - N), (0, 0)))
    zeros_np = jnp.zeros((NP, H), jnp.float32)

    h = _matmul_bias(xp, Wn, bn, NB)
    e = [_matmul_bias(edge_attr[offs[p]:offs[p + 1]], We, be, EBLK)
         for p in range(P)]

    Whp = jnp.pad(Wh, ((0, 0), (0, H - Wh.shape[1])))
    bhp = jnp.pad(bh, (0, H - bh.shape[0])).reshape(1, H)

    pred = None
    for i in range(L):
        parts = [_msg_scatter(h, e[p], srcm[p], dstm[p], zeros_np,
                              epws[p], msg_stages[p], SCAP)
                 for p in range(P)]
        epsb = jnp.full((1, H), 1.0, jnp.float32) + eps[i]
        z, s, s2 = _node_z(h, parts[0], parts[1],
                           W1[i], b1[i], W2[i], b2[i], epsb)
        if i < L - 1:
            h, A, B = _node_bn_ab(z, h, s, s2, gamma[i], beta[i],
                                  We1[i, :H], We1[i, H:2 * H])
            G = [_gather_ab(A, B, srcg[p], dstg[p], EPARTS[p],
                            epws[p], nchg[p]) for p in range(P)]
            e = [_edge_mlp(e[p], G[p], We1[i, 2 * H:], be1[i],
                           We2[i], be2[i]) for p in range(P)]
        else:
            pred = _node_bn_head(z, h, s, s2, gamma[i], beta[i], Whp, bhp)

    return pred[:N, :1]
